# baseline (speedup 1.0000x reference)
import functools

import jax
import jax.numpy as jnp
from jax import lax
from jax.experimental import pallas as pl
from jax.experimental.pallas import tpu as pltpu
from jax.experimental.pallas import tpu_sc as plsc

NC = 2
NS = 16
NW = NC * NS
CHUNK = 128


def _body(idx_hbm, table_hbm, out_hbm, idx_v, rows_v, out_v, gsem, wsem):
    wid = lax.axis_index("s") * NC + lax.axis_index("c")
    n_chunks = idx_hbm.shape[1]
    rows_per_w = n_chunks * CHUNK

    pltpu.sync_copy(idx_hbm.at[wid], idx_v)

    iota = lax.iota(jnp.int32, 16)
    ce0 = iota * 2
    co0 = ce0 + 1
    ce1 = ce0 + 32
    co1 = co0 + 32
    base_row = wid * rows_per_w

    @pl.loop(0, n_chunks)
    def _chunk(k):
        pltpu.async_copy(table_hbm.at[idx_v.at[k]], rows_v, gsem).wait()

        @pl.loop(0, CHUNK, unroll=8)
        def _row(t):
            rvec = jnp.broadcast_to(t, (16,))
            ea0 = plsc.load_gather(rows_v, [rvec, ce0])
            eb0 = plsc.load_gather(rows_v, [rvec, co0])
            out_v[t, pl.ds(0, 32)] = plsc.pack(
                ea0, eb0, format=plsc.PackFormat.INTERLEAVED)
            ea1 = plsc.load_gather(rows_v, [rvec, ce1])
            eb1 = plsc.load_gather(rows_v, [rvec, co1])
            out_v[t, pl.ds(32, 32)] = plsc.pack(
                ea1, eb1, format=plsc.PackFormat.INTERLEAVED)

        pltpu.async_copy(
            out_v, out_hbm.at[pl.ds(base_row + k * CHUNK, CHUNK)], wsem
        ).wait()


def kernel(x, weight):
    batch, hist = x.shape
    n_total = batch * hist
    d = weight.shape[1]
    assert n_total % (NW * CHUNK) == 0
    n_chunks = n_total // (NW * CHUNK)
    idx = x.reshape(NW, n_chunks, CHUNK)

    run = functools.partial(
        pl.kernel,
        out_type=jax.ShapeDtypeStruct((n_total, d), jnp.bfloat16),
        mesh=plsc.VectorSubcoreMesh(core_axis_name="c", subcore_axis_name="s"),
        compiler_params=pltpu.CompilerParams(
            needs_layout_passes=False, use_tc_tiling_on_sc=False),
        scratch_types=[
            pltpu.VMEM((n_chunks, CHUNK), jnp.int32),
            pltpu.VMEM((CHUNK, d), jnp.float32),
            pltpu.VMEM((CHUNK, d), jnp.bfloat16),
            pltpu.SemaphoreType.DMA,
            pltpu.SemaphoreType.DMA,
        ],
    )(_body)
    out = run(idx, weight)
    return out.reshape(batch, hist, d)


# speedup vs baseline: 1.1616x; 1.1616x over previous
import functools

import jax
import jax.numpy as jnp
from jax import lax
from jax.experimental import pallas as pl
from jax.experimental.pallas import tpu as pltpu
from jax.experimental.pallas import tpu_sc as plsc

NC = 2
NS = 16
NW = NC * NS
GATHER = 128
SUB = 2
CHUNK = GATHER * SUB


def _body(idx_hbm, table_hbm, out_hbm, idx_v, rows0, rows1, out0, out1,
          gsem, wsem):
    wid = lax.axis_index("s") * NC + lax.axis_index("c")
    n_chunks = idx_hbm.shape[1] // SUB
    rows_per_w = n_chunks * CHUNK
    base_row = wid * rows_per_w

    pltpu.sync_copy(idx_hbm.at[wid], idx_v)

    iota = lax.iota(jnp.int32, 16)
    ce0 = iota * 2
    co0 = ce0 + 1
    ce1 = ce0 + 32
    co1 = co0 + 32

    def gather(c, rows, start):
        for h in range(SUB):
            cp = pltpu.make_async_copy(
                table_hbm.at[idx_v.at[SUB * c + h]],
                rows.at[pl.ds(h * GATHER, GATHER)], gsem)
            cp.start() if start else cp.wait()

    def write(c, out, start):
        cp = pltpu.make_async_copy(
            out, out_hbm.at[pl.ds(base_row + c * CHUNK, CHUNK)], wsem)
        cp.start() if start else cp.wait()

    gather(0, rows0, True)
    gather(1, rows1, True)

    @pl.loop(0, n_chunks, step=2)
    def _super(kk):
        for rows, out, b in ((rows0, out0, 0), (rows1, out1, 1)):
            k = kk + b
            gather(k, rows, False)

            @pl.when(k >= 2)
            def _():
                write(k - 2, out, False)

            @pl.loop(0, CHUNK, unroll=8)
            def _row(t):
                rvec = jnp.broadcast_to(t, (16,))
                ea0 = plsc.load_gather(rows, [rvec, ce0])
                eb0 = plsc.load_gather(rows, [rvec, co0])
                out[t, pl.ds(0, 32)] = plsc.pack(
                    ea0, eb0, format=plsc.PackFormat.INTERLEAVED)
                ea1 = plsc.load_gather(rows, [rvec, ce1])
                eb1 = plsc.load_gather(rows, [rvec, co1])
                out[t, pl.ds(32, 32)] = plsc.pack(
                    ea1, eb1, format=plsc.PackFormat.INTERLEAVED)

            write(k, out, True)

            @pl.when(k + 2 < n_chunks)
            def _():
                gather(k + 2, rows, True)

    write(n_chunks - 2, out0, False)
    write(n_chunks - 1, out1, False)


def kernel(x, weight):
    batch, hist = x.shape
    n_total = batch * hist
    d = weight.shape[1]
    assert n_total % (NW * CHUNK) == 0
    n_gathers = n_total // (NW * GATHER)
    idx = x.reshape(NW, n_gathers, GATHER)

    run = functools.partial(
        pl.kernel,
        out_type=jax.ShapeDtypeStruct((n_total, d), jnp.bfloat16),
        mesh=plsc.VectorSubcoreMesh(core_axis_name="c", subcore_axis_name="s"),
        compiler_params=pltpu.CompilerParams(
            needs_layout_passes=False, use_tc_tiling_on_sc=False),
        scratch_types=[
            pltpu.VMEM((n_gathers, GATHER), jnp.int32),
            pltpu.VMEM((CHUNK, d), jnp.float32),
            pltpu.VMEM((CHUNK, d), jnp.float32),
            pltpu.VMEM((CHUNK, d), jnp.bfloat16),
            pltpu.VMEM((CHUNK, d), jnp.bfloat16),
            pltpu.SemaphoreType.DMA,
            pltpu.SemaphoreType.DMA,
        ],
    )(_body)
    out = run(idx, weight)
    return out.reshape(batch, hist, d)
